# baseline (speedup 1.0000x reference)
import functools

import jax
import jax.numpy as jnp
from jax import lax
from jax.experimental import pallas as pl
from jax.experimental.pallas import tpu as pltpu
from jax.experimental.pallas import tpu_sc as plsc

NUM_BLOCKS = 8
M = 2000
PM = 2048
D = 2048
NC = 2
CS = 200
PER_BLK = M // CS
ITERS = 4 * PER_BLK
NBUF = 2


def _unpad(inp):
    mesh = plsc.VectorSubcoreMesh(core_axis_name="c", subcore_axis_name="s")

    @functools.partial(
        pl.kernel,
        mesh=mesh,
        out_type=jax.ShapeDtypeStruct((NUM_BLOCKS * M, D), jnp.float32),
        scratch_types=(
            [pltpu.VMEM_SHARED((CS, D), jnp.float32)] * NBUF
            + [pltpu.SemaphoreType.DMA] * (2 * NBUF)
        ),
    )
    def k(inp_hbm, out_hbm, *scr):
        bufs = scr[:NBUF]
        isems = scr[NBUF : 2 * NBUF]
        osems = scr[2 * NBUF :]
        core = lax.axis_index("c")
        sid = lax.axis_index("s")

        @pl.when(sid == 0)
        def _work():
            def base(i):
                blk = core * 4 + (i // PER_BLK)
                j = i % PER_BLK
                return blk, j * CS

            def start_in(i):
                slot = i % NBUF
                blk, off = base(i)
                s = pl.multiple_of(blk * PM + off, 8)
                return pltpu.async_copy(
                    inp_hbm.at[pl.ds(s, CS), :], bufs[slot], isems[slot]
                )

            def start_out(i):
                slot = i % NBUF
                blk, off = base(i)
                d = pl.multiple_of(blk * M + off, 8)
                return pltpu.async_copy(
                    bufs[slot], out_hbm.at[pl.ds(d, CS), :], osems[slot]
                )

            in_h = {}
            out_h = {}
            for i in range(NBUF - 1):
                in_h[i] = start_in(i)
            for i in range(ITERS):
                if i not in in_h:
                    in_h[i] = start_in(i)
                in_h[i].wait()
                out_h[i] = start_out(i)
                j = i + NBUF - 1
                if j < ITERS and j not in in_h:
                    if j - NBUF >= 0:
                        out_h[j - NBUF].wait()
                    in_h[j] = start_in(j)
            for i in range(max(0, ITERS - NBUF), ITERS):
                out_h[i].wait()

    return k(inp)


def kernel(inp, m_splits):
    inp2d = inp.reshape(-1, inp.shape[-1])
    return _unpad(inp2d)


# speedup vs baseline: 1.1401x; 1.1401x over previous
import functools

import jax
import jax.numpy as jnp
from jax import lax
from jax.experimental import pallas as pl
from jax.experimental.pallas import tpu as pltpu
from jax.experimental.pallas import tpu_sc as plsc

NUM_BLOCKS = 8
M = 2000
PM = 2048
D = 2048
NC = 2
NS = 16
NSW = NC * (NS - 1)

SB = 1080
C = 24
TPB = SB // C
TASKS_PER_W = (NUM_BLOCKS * TPB) // NSW
S_NBUF = 2

SPB = M - SB
CS = 120
SP_BASES = (0, 120, 240, 360, 480, 600, 720, 800)
SP_ITERS = 4 * len(SP_BASES)
P_NBUF = 2


def _ring(iters, nbuf, start_in, start_out):
    in_h = {}
    out_h = {}
    for i in range(min(nbuf - 1, iters)):
        in_h[i] = start_in(i)
    for i in range(iters):
        if i not in in_h:
            in_h[i] = start_in(i)
        in_h[i].wait()
        out_h[i] = start_out(i)
        j = i + nbuf - 1
        if j < iters and j not in in_h:
            if j - nbuf >= 0:
                out_h[j - nbuf].wait()
            in_h[j] = start_in(j)
    for i in range(max(0, iters - nbuf), iters):
        out_h[i].wait()


def _unpad(inp):
    mesh = plsc.VectorSubcoreMesh(core_axis_name="c", subcore_axis_name="s")

    @functools.partial(
        pl.kernel,
        mesh=mesh,
        out_type=jax.ShapeDtypeStruct((NUM_BLOCKS * M, D), jnp.float32),
        scratch_types=(
            [pltpu.VMEM((C, D), jnp.float32)] * S_NBUF
            + [pltpu.VMEM_SHARED((CS, D), jnp.float32)] * P_NBUF
            + [pltpu.SemaphoreType.DMA] * (2 * S_NBUF + 2 * P_NBUF)
        ),
    )
    def k(inp_hbm, out_hbm, *scr):
        sbufs = scr[:S_NBUF]
        pbufs = scr[S_NBUF : S_NBUF + P_NBUF]
        sems = scr[S_NBUF + P_NBUF :]
        s_isems = sems[:S_NBUF]
        s_osems = sems[S_NBUF : 2 * S_NBUF]
        p_isems = sems[2 * S_NBUF : 2 * S_NBUF + P_NBUF]
        p_osems = sems[2 * S_NBUF + P_NBUF :]
        core = lax.axis_index("c")
        sid = lax.axis_index("s")

        @pl.when(sid < NS - 1)
        def _stream():
            ws = core * (NS - 1) + sid

            def rows(i):
                t = ws * TASKS_PER_W + i
                blk = t // TPB
                off = (t % TPB) * C
                return blk, off

            def start_in(i):
                slot = i % S_NBUF
                blk, off = rows(i)
                s = pl.multiple_of(blk * PM + off, 8)
                return pltpu.async_copy(
                    inp_hbm.at[pl.ds(s, C), :], sbufs[slot], s_isems[slot]
                )

            def start_out(i):
                slot = i % S_NBUF
                blk, off = rows(i)
                d = pl.multiple_of(blk * M + off, 8)
                return pltpu.async_copy(
                    sbufs[slot], out_hbm.at[pl.ds(d, C), :], s_osems[slot]
                )

            _ring(TASKS_PER_W, S_NBUF, start_in, start_out)

        @pl.when(sid == NS - 1)
        def _spmem_manager():
            def rows(i):
                blk = core * 4 + (i // len(SP_BASES))
                off = SB + SP_BASES[i % len(SP_BASES)]
                return blk, off

            def start_in(i):
                slot = i % P_NBUF
                blk, off = rows(i)
                s = pl.multiple_of(blk * PM + off, 8)
                return pltpu.async_copy(
                    inp_hbm.at[pl.ds(s, CS), :], pbufs[slot], p_isems[slot]
                )

            def start_out(i):
                slot = i % P_NBUF
                blk, off = rows(i)
                d = pl.multiple_of(blk * M + off, 8)
                return pltpu.async_copy(
                    pbufs[slot], out_hbm.at[pl.ds(d, CS), :], p_osems[slot]
                )

            _ring(SP_ITERS, P_NBUF, start_in, start_out)

    return k(inp)


def kernel(inp, m_splits):
    inp2d = inp.reshape(-1, inp.shape[-1])
    return _unpad(inp2d)
